# baseline (speedup 1.0000x reference)
import jax
import jax.numpy as jnp
from jax import lax
from jax.experimental import pallas as pl
from jax.experimental.pallas import tpu as pltpu
from jax.experimental.pallas import tpu_sc as plsc

BATCH = 16384
N_FIELDS = 26
EMBED_DIM = 32

NC, NS = 2, 16
NW = NC * NS
ROWS_PER_W = BATCH // NW
CHUNK = 32
N_CHUNKS = ROWS_PER_W // CHUNK
FPAD = 32
IDX_W = 128
GATHERS = CHUNK * FPAD // IDX_W
HALVES = EMBED_DIM // 16

_SEG = 249984
_PACKED_ROWS = 250048
_TB4 = 8064

_OBC = 1024


_GATHER_DNUMS = lax.GatherDimensionNumbers(
    offset_dims=(), collapsed_slice_dims=(0,), start_index_map=(0,))


def _shuffle16(v, idx):
    return lax.gather(v, idx[:, None], _GATHER_DNUMS, (1,),
                      mode=lax.GatherScatterMode.PROMISE_IN_BOUNDS)


def _lane_sum(v):
    lanes = lax.iota(jnp.int32, 16)
    for s in (8, 4, 2, 1):
        v = v + _shuffle16(v, lanes ^ s)
    return v


def _fast_rsqrt(v):
    i = lax.bitcast_convert_type(v, jnp.int32)
    i = jnp.int32(0x5F3759DF) - (i >> 1)
    y = lax.bitcast_convert_type(i, jnp.float32)
    for _ in range(3):
        y = y * (1.5 - 0.5 * v * y * y)
    return y


def _sc_body(x_hbm, table_hbm, out_hbm, idx_v, rows_a, rows_b,
             gsem_a, gsem_b, wsem_a, wsem_b):
    wid = lax.axis_index("s") * NC + lax.axis_index("c")

    pltpu.sync_copy(x_hbm.at[wid], idx_v)

    def remap_body(r, carry):
        for h in range(IDX_W // 16):
            sl = (r, pl.ds(h * 16, 16))
            v = idx_v[sl]
            one = jnp.ones((16,), jnp.int32)
            zero = jnp.zeros((16,), jnp.int32)
            a = (jnp.where(v >= _SEG, one, zero)
                 + jnp.where(v >= 2 * _SEG, one, zero)
                 + jnp.where(v >= 3 * _SEG, one, zero))
            idx_v[sl] = 4 * (v - a * _SEG) + a
        return carry

    lax.fori_loop(0, idx_v.shape[0], remap_body, 0)

    def fire(c, rows, gsem):
        return [
            pltpu.async_copy(
                table_hbm.at[idx_v.at[c * GATHERS + j]],
                rows.at[pl.ds(j * IDX_W, IDX_W)],
                gsem,
            )
            for j in range(GATHERS)
        ]

    def drain_gathers(rows, gsem):
        for j in range(GATHERS):
            pltpu.make_async_copy(
                table_hbm.at[pl.ds(0, IDX_W)],
                rows.at[pl.ds(j * IDX_W, IDX_W)],
                gsem,
            ).wait()

    def drain_wb(rows, wsem):
        pltpu.make_async_copy(
            rows, out_hbm.at[pl.ds(0, CHUNK * FPAD)], wsem).wait()

    def out_slice(c):
        row0 = (wid * ROWS_PER_W + c * CHUNK) * FPAD
        return out_hbm.at[pl.ds(row0, CHUNK * FPAD)]

    def compute(rows):
        def row_body(i, carry2):
            base = i * FPAD
            acc = jnp.zeros((16,), jnp.float32)
            for r in range(N_FIELDS):
                for h in range(HALVES):
                    v = rows[base + r, pl.ds(h * 16, 16)]
                    acc = acc + v * v
            ssq = jnp.maximum(_lane_sum(acc), 1e-24)
            scale = _fast_rsqrt(ssq)
            for r in range(N_FIELDS):
                for h in range(HALVES):
                    sl = (base + r, pl.ds(h * 16, 16))
                    rows[sl] = rows[sl] * scale
            return carry2

        lax.fori_loop(0, CHUNK, row_body, 0)

    fire(0, rows_a, gsem_a)

    def pair_body(k, carry):
        c0 = 2 * k
        drain_gathers(rows_a, gsem_a)

        @pl.when(k > 0)
        def _():
            drain_wb(rows_b, wsem_b)

        hb = fire(c0 + 1, rows_b, gsem_b)
        compute(rows_a)
        wa = pltpu.async_copy(rows_a, out_slice(c0), wsem_a)
        for cp in hb:
            cp.wait()
        wa.wait()

        @pl.when(k + 1 < N_CHUNKS // 2)
        def _():
            fire(c0 + 2, rows_a, gsem_a)

        compute(rows_b)
        pltpu.async_copy(rows_b, out_slice(c0 + 1), wsem_b)
        return carry

    lax.fori_loop(0, N_CHUNKS // 2, pair_body, 0)
    drain_wb(rows_b, wsem_b)


def _transpose_body(t0, t1, t2, t3, out_ref):
    m = jnp.concatenate([t0[...], t1[...], t2[...], t3[...]], axis=0)
    out_ref[...] = m.T


def _tc_transpose_table(table):
    tt = jnp.swapaxes(table, 0, 1)
    step = _SEG // _TB4
    grid = (_PACKED_ROWS + _TB4 - 1) // _TB4
    packed = pl.pallas_call(
        _transpose_body,
        grid=(grid,),
        in_specs=[
            pl.BlockSpec((EMBED_DIM, _TB4),
                         lambda i, a=a: (0, a * step + i))
            for a in range(4)
        ],
        out_specs=pl.BlockSpec((_TB4, 128), lambda i: (i, 0)),
        out_shape=jax.ShapeDtypeStruct((_PACKED_ROWS, 128), table.dtype),
    )(tt, tt, tt, tt)
    return packed.reshape(_PACKED_ROWS * 4, EMBED_DIM)


def _out_transpose_body(in3_ref, o_ref):
    for q in range(7):
        t = in3_ref[:, q, :].T
        if q < 6:
            o_ref[q * 128:(q + 1) * 128, :] = t
        else:
            o_ref[768:832, :] = t[:64, :]


def _tc_transpose_out(flat):
    grid = BATCH // _OBC
    return pl.pallas_call(
        _out_transpose_body,
        grid=(grid,),
        in_specs=[pl.BlockSpec((_OBC, 8, 128), lambda i: (i, 0, 0))],
        out_specs=pl.BlockSpec((N_FIELDS * EMBED_DIM, _OBC),
                               lambda i: (0, i)),
        out_shape=jax.ShapeDtypeStruct((N_FIELDS * EMBED_DIM, BATCH),
                                       jnp.float32),
    )(flat)


def kernel(x, table):
    table = _tc_transpose_table(table)
    xp = jnp.pad(x, ((0, 0), (0, FPAD - N_FIELDS)))
    x2 = xp.reshape(NW, ROWS_PER_W * FPAD // IDX_W, IDX_W)
    out = pl.kernel(
        _sc_body,
        out_type=jax.ShapeDtypeStruct((BATCH * FPAD, EMBED_DIM),
                                      jnp.float32),
        mesh=plsc.VectorSubcoreMesh(core_axis_name="c", subcore_axis_name="s"),
        compiler_params=pltpu.CompilerParams(use_tc_tiling_on_sc=False),
        scratch_types=[
            pltpu.VMEM((ROWS_PER_W * FPAD // IDX_W, IDX_W), jnp.int32),
            pltpu.VMEM((CHUNK * FPAD, EMBED_DIM), jnp.float32),
            pltpu.VMEM((CHUNK * FPAD, EMBED_DIM), jnp.float32),
            pltpu.SemaphoreType.DMA,
            pltpu.SemaphoreType.DMA,
            pltpu.SemaphoreType.DMA,
            pltpu.SemaphoreType.DMA,
        ],
    )(x2, table)
    ocm = _tc_transpose_out(out.reshape(BATCH, 8, 128))
    return ocm.T


# speedup vs baseline: 5.2544x; 5.2544x over previous
import jax
import jax.numpy as jnp
from jax import lax
from jax.experimental import pallas as pl
from jax.experimental.pallas import tpu as pltpu
from jax.experimental.pallas import tpu_sc as plsc

BATCH = 16384
N_FIELDS = 26
EMBED_DIM = 32
VOCAB = 1000000

NC, NS = 2, 16
NW = NC * NS
ROWS_PER_W = BATCH // NW
CHUNK = 32
N_CHUNKS = ROWS_PER_W // CHUNK
FPAD = 32
IDX_W = 128
GATHERS = CHUNK * FPAD // IDX_W
HALVES = EMBED_DIM // 16

_SEG = 249984
_PACKED_ROWS = 250048
_TB4 = 8064

_OBC = 1024


_GATHER_DNUMS = lax.GatherDimensionNumbers(
    offset_dims=(), collapsed_slice_dims=(0,), start_index_map=(0,))


def _shuffle16(v, idx):
    return lax.gather(v, idx[:, None], _GATHER_DNUMS, (1,),
                      mode=lax.GatherScatterMode.PROMISE_IN_BOUNDS)


def _lane_sum(v):
    lanes = lax.iota(jnp.int32, 16)
    for s in (8, 4, 2, 1):
        v = v + _shuffle16(v, lanes ^ s)
    return v


def _fast_rsqrt(v):
    i = lax.bitcast_convert_type(v, jnp.int32)
    i = jnp.int32(0x5F3759DF) - (i >> 1)
    y = lax.bitcast_convert_type(i, jnp.float32)
    for _ in range(3):
        y = y * (1.5 - 0.5 * v * y * y)
    return y


def _sc_body(x_hbm, table_hbm, out_hbm, idx_v, rows_a, rows_b, wb_scratch,
             gsem_a, gsem_b, wsem_a, wsem_b):
    wid = lax.axis_index("s") * NC + lax.axis_index("c")

    pltpu.sync_copy(x_hbm.at[wid], idx_v)

    def remap_body(r, carry):
        for h in range(IDX_W // 16):
            sl = (r, pl.ds(h * 16, 16))
            v = idx_v[sl]
            one = jnp.ones((16,), jnp.int32)
            zero = jnp.zeros((16,), jnp.int32)
            a = (jnp.where(v >= _SEG, one, zero)
                 + jnp.where(v >= 2 * _SEG, one, zero)
                 + jnp.where(v >= 3 * _SEG, one, zero))
            idx_v[sl] = 4 * (v - a * _SEG) + a
        return carry

    lax.fori_loop(0, idx_v.shape[0], remap_body, 0)

    def fire(c, rows, gsem):
        return [
            pltpu.async_copy(
                table_hbm.at[
                    idx_v.at[jnp.minimum(c * GATHERS + j,
                                         ROWS_PER_W * FPAD // IDX_W - 1)]],
                rows.at[pl.ds(j * IDX_W, IDX_W)],
                gsem,
            )
            for j in range(GATHERS)
        ]

    def drain_gathers(rows, gsem):
        pltpu.make_async_copy(
            table_hbm.at[pl.ds(0, CHUNK * FPAD)], rows, gsem).wait()

    def drain_wb(rows, wsem):
        pltpu.make_async_copy(
            rows, out_hbm.at[pl.ds(0, CHUNK * FPAD)], wsem).wait()

    def out_slice(c):
        row0 = (wid * ROWS_PER_W + c * CHUNK) * FPAD
        return out_hbm.at[pl.ds(row0, CHUNK * FPAD)]

    def compute(rows):
        def row_body(i, carry2):
            base = i * FPAD
            acc = jnp.zeros((16,), jnp.float32)
            for r in range(N_FIELDS):
                for h in range(HALVES):
                    v = rows[base + r, pl.ds(h * 16, 16)]
                    acc = acc + v * v
            ssq = jnp.maximum(_lane_sum(acc), 1e-24)
            scale = _fast_rsqrt(ssq)
            for r in range(N_FIELDS):
                for h in range(HALVES):
                    sl = (base + r, pl.ds(h * 16, 16))
                    rows[sl] = rows[sl] * scale
            return carry2

        lax.fori_loop(0, CHUNK, row_body, 0)

    fire(0, rows_a, gsem_a)
    pltpu.async_copy(rows_b, wb_scratch, wsem_b)

    def pair_body(k, carry):
        c0 = 2 * k
        drain_gathers(rows_a, gsem_a)
        drain_wb(rows_b, wsem_b)
        hb = fire(c0 + 1, rows_b, gsem_b)
        compute(rows_a)
        wa = pltpu.async_copy(rows_a, out_slice(c0), wsem_a)
        for cp in hb:
            cp.wait()
        wa.wait()
        fire(c0 + 2, rows_a, gsem_a)
        compute(rows_b)
        pltpu.async_copy(rows_b, out_slice(c0 + 1), wsem_b)
        return carry

    lax.fori_loop(0, N_CHUNKS // 2, pair_body, 0)
    drain_wb(rows_b, wsem_b)
    drain_gathers(rows_a, gsem_a)


def _transpose_body(t0, t1, t2, t3, out_ref):
    m = jnp.concatenate([t0[...], t1[...], t2[...], t3[...]], axis=0)
    out_ref[...] = m.T


def _tc_transpose_table(table):
    tt = jnp.swapaxes(table, 0, 1)
    step = _SEG // _TB4
    grid = (_PACKED_ROWS + _TB4 - 1) // _TB4
    packed = pl.pallas_call(
        _transpose_body,
        grid=(grid,),
        in_specs=[
            pl.BlockSpec((EMBED_DIM, _TB4),
                         lambda i, a=a: (0, a * step + i))
            for a in range(4)
        ],
        out_specs=pl.BlockSpec((_TB4, 128), lambda i: (i, 0)),
        out_shape=jax.ShapeDtypeStruct((_PACKED_ROWS, 128), table.dtype),
    )(tt, tt, tt, tt)
    return packed.reshape(_PACKED_ROWS * 4, EMBED_DIM)


def _out_transpose_body(in3_ref, o_ref):
    for q in range(7):
        t = in3_ref[:, q, :].T
        if q < 6:
            o_ref[q * 128:(q + 1) * 128, :] = t
        else:
            o_ref[768:832, :] = t[:64, :]


def _tc_transpose_out(flat):
    grid = BATCH // _OBC
    return pl.pallas_call(
        _out_transpose_body,
        grid=(grid,),
        in_specs=[pl.BlockSpec((_OBC, 8, 128), lambda i: (i, 0, 0))],
        out_specs=pl.BlockSpec((N_FIELDS * EMBED_DIM, _OBC),
                               lambda i: (0, i)),
        out_shape=jax.ShapeDtypeStruct((N_FIELDS * EMBED_DIM, BATCH),
                                       jnp.float32),
    )(flat)


def kernel(x, table):
    table = _tc_transpose_table(table)
    aux = ((61 * jnp.arange(BATCH, dtype=jnp.int32))[:, None]
           + 16384 * jnp.arange(FPAD - N_FIELDS, dtype=jnp.int32)[None, :]
           ) % VOCAB
    xp = jnp.concatenate([x, aux], axis=1)
    x2 = xp.reshape(NW, ROWS_PER_W * FPAD // IDX_W, IDX_W)
    out = pl.kernel(
        _sc_body,
        out_type=jax.ShapeDtypeStruct((BATCH * FPAD, EMBED_DIM),
                                      jnp.float32),
        mesh=plsc.VectorSubcoreMesh(core_axis_name="c", subcore_axis_name="s"),
        compiler_params=pltpu.CompilerParams(use_tc_tiling_on_sc=False),
        scratch_types=[
            pltpu.VMEM((ROWS_PER_W * FPAD // IDX_W, IDX_W), jnp.int32),
            pltpu.VMEM((CHUNK * FPAD, EMBED_DIM), jnp.float32),
            pltpu.VMEM((CHUNK * FPAD, EMBED_DIM), jnp.float32),
            pltpu.HBM((CHUNK * FPAD, EMBED_DIM), jnp.float32),
            pltpu.SemaphoreType.DMA,
            pltpu.SemaphoreType.DMA,
            pltpu.SemaphoreType.DMA,
            pltpu.SemaphoreType.DMA,
        ],
    )(x2, table)
    ocm = _tc_transpose_out(out.reshape(BATCH, 8, 128))
    return ocm.T


# speedup vs baseline: 5.2704x; 1.0030x over previous
import jax
import jax.numpy as jnp
from jax import lax
from jax.experimental import pallas as pl
from jax.experimental.pallas import tpu as pltpu
from jax.experimental.pallas import tpu_sc as plsc

BATCH = 16384
N_FIELDS = 26
EMBED_DIM = 32
VOCAB = 1000000

NC, NS = 2, 16
NW = NC * NS
ROWS_PER_W = BATCH // NW
HALF_ROWS = ROWS_PER_W // 2
CHUNK = 32
N_CHUNKS_H = HALF_ROWS // CHUNK
FPAD = 32
IDX_W = 128
GATHERS = CHUNK * FPAD // IDX_W
HALVES = EMBED_DIM // 16

_SEG = 249984
_PACKED_ROWS = 250048
_TB4 = 8064

_OBC = 1024


_GATHER_DNUMS = lax.GatherDimensionNumbers(
    offset_dims=(), collapsed_slice_dims=(0,), start_index_map=(0,))


def _shuffle16(v, idx):
    return lax.gather(v, idx[:, None], _GATHER_DNUMS, (1,),
                      mode=lax.GatherScatterMode.PROMISE_IN_BOUNDS)


def _lane_sum(v):
    lanes = lax.iota(jnp.int32, 16)
    for s in (8, 4, 2, 1):
        v = v + _shuffle16(v, lanes ^ s)
    return v


def _fast_rsqrt(v):
    i = lax.bitcast_convert_type(v, jnp.int32)
    i = jnp.int32(0x5F3759DF) - (i >> 1)
    y = lax.bitcast_convert_type(i, jnp.float32)
    for _ in range(3):
        y = y * (1.5 - 0.5 * v * y * y)
    return y


def _sc_body(half, x_hbm, table_hbm, out_hbm, idx_v, rows_a, rows_b,
             wb_scratch, gsem_a, gsem_b, wsem_a, wsem_b):
    wid = lax.axis_index("s") * NC + lax.axis_index("c")
    idx_rows = HALF_ROWS * FPAD // IDX_W

    pltpu.sync_copy(x_hbm.at[wid, pl.ds(half * idx_rows, idx_rows)], idx_v)

    def remap_body(r, carry):
        for h in range(IDX_W // 16):
            sl = (r, pl.ds(h * 16, 16))
            v = idx_v[sl]
            one = jnp.ones((16,), jnp.int32)
            zero = jnp.zeros((16,), jnp.int32)
            a = (jnp.where(v >= _SEG, one, zero)
                 + jnp.where(v >= 2 * _SEG, one, zero)
                 + jnp.where(v >= 3 * _SEG, one, zero))
            idx_v[sl] = 4 * (v - a * _SEG) + a
        return carry

    lax.fori_loop(0, idx_v.shape[0], remap_body, 0)

    def fire(c, rows, gsem):
        return [
            pltpu.async_copy(
                table_hbm.at[
                    idx_v.at[jnp.minimum(c * GATHERS + j, idx_rows - 1)]],
                rows.at[pl.ds(j * IDX_W, IDX_W)],
                gsem,
            )
            for j in range(GATHERS)
        ]

    def drain_gathers(rows, gsem):
        pltpu.make_async_copy(
            table_hbm.at[pl.ds(0, CHUNK * FPAD)], rows, gsem).wait()

    def drain_wb(rows, wsem):
        pltpu.make_async_copy(
            rows, out_hbm.at[pl.ds(0, CHUNK * FPAD)], wsem).wait()

    def out_slice(c):
        row0 = (wid * HALF_ROWS + c * CHUNK) * FPAD
        return out_hbm.at[pl.ds(row0, CHUNK * FPAD)]

    def compute(rows):
        def row_body(i, carry2):
            base = i * FPAD
            acc = jnp.zeros((16,), jnp.float32)
            for r in range(N_FIELDS):
                for h in range(HALVES):
                    v = rows[base + r, pl.ds(h * 16, 16)]
                    acc = acc + v * v
            ssq = jnp.maximum(_lane_sum(acc), 1e-24)
            scale = _fast_rsqrt(ssq)
            for r in range(N_FIELDS):
                for h in range(HALVES):
                    sl = (base + r, pl.ds(h * 16, 16))
                    rows[sl] = rows[sl] * scale
            return carry2

        lax.fori_loop(0, CHUNK, row_body, 0)

    fire(0, rows_a, gsem_a)
    pltpu.async_copy(rows_b, wb_scratch, wsem_b)

    def pair_body(k, carry):
        c0 = 2 * k
        drain_gathers(rows_a, gsem_a)
        drain_wb(rows_b, wsem_b)
        hb = fire(c0 + 1, rows_b, gsem_b)
        compute(rows_a)
        wa = pltpu.async_copy(rows_a, out_slice(c0), wsem_a)
        for cp in hb:
            cp.wait()
        wa.wait()
        fire(c0 + 2, rows_a, gsem_a)
        compute(rows_b)
        pltpu.async_copy(rows_b, out_slice(c0 + 1), wsem_b)
        return carry

    lax.fori_loop(0, N_CHUNKS_H // 2, pair_body, 0)
    drain_wb(rows_b, wsem_b)
    drain_gathers(rows_a, gsem_a)


def _transpose_body(t0, t1, t2, t3, out_ref):
    m = jnp.concatenate([t0[...], t1[...], t2[...], t3[...]], axis=0)
    out_ref[...] = m.T


def _tc_transpose_table(table):
    tt = jnp.swapaxes(table, 0, 1)
    step = _SEG // _TB4
    grid = (_PACKED_ROWS + _TB4 - 1) // _TB4
    packed = pl.pallas_call(
        _transpose_body,
        grid=(grid,),
        in_specs=[
            pl.BlockSpec((EMBED_DIM, _TB4),
                         lambda i, a=a: (0, a * step + i))
            for a in range(4)
        ],
        out_specs=pl.BlockSpec((_TB4, 128), lambda i: (i, 0)),
        out_shape=jax.ShapeDtypeStruct((_PACKED_ROWS, 128), table.dtype),
    )(tt, tt, tt, tt)
    return packed.reshape(_PACKED_ROWS * 4, EMBED_DIM)


def _out_transpose_body(in3_ref, o_ref):
    for q in range(7):
        t = in3_ref[:, q, :].T
        if q < 6:
            o_ref[q * 128:(q + 1) * 128, :] = t
        else:
            o_ref[768:832, :] = t[:64, :]


def _out_transpose_body2(in3_ref, prev_ref, o_ref):
    del prev_ref
    _out_transpose_body(in3_ref, o_ref)


def _tc_transpose_out(flat, half, prev=None):
    grid = BATCH // 2 // _OBC
    blocks_per_half = grid
    in_spec = pl.BlockSpec((_OBC, 8, 128), lambda i: (i, 0, 0))
    out_spec = pl.BlockSpec(
        (N_FIELDS * EMBED_DIM, _OBC),
        lambda i, half=half: (0, i + half * blocks_per_half))
    out_shape = jax.ShapeDtypeStruct((N_FIELDS * EMBED_DIM, BATCH),
                                     jnp.float32)
    if prev is None:
        return pl.pallas_call(
            _out_transpose_body, grid=(grid,),
            in_specs=[in_spec], out_specs=out_spec, out_shape=out_shape,
        )(flat)
    return pl.pallas_call(
        _out_transpose_body2, grid=(grid,),
        in_specs=[in_spec,
                  pl.BlockSpec(memory_space=pl.ANY)],
        out_specs=out_spec, out_shape=out_shape,
        input_output_aliases={1: 0},
    )(flat, prev)


def kernel(x, table):
    table = _tc_transpose_table(table)
    aux = ((61 * jnp.arange(BATCH, dtype=jnp.int32))[:, None]
           + 16384 * jnp.arange(FPAD - N_FIELDS, dtype=jnp.int32)[None, :]
           ) % VOCAB
    xp = jnp.concatenate([x, aux], axis=1)
    x2 = xp.reshape(NW, ROWS_PER_W * FPAD // IDX_W, IDX_W)

    def sc_half(half):
        import functools
        return pl.kernel(
            functools.partial(_sc_body, half),
            out_type=jax.ShapeDtypeStruct((BATCH // 2 * FPAD, EMBED_DIM),
                                          jnp.float32),
            mesh=plsc.VectorSubcoreMesh(core_axis_name="c",
                                        subcore_axis_name="s"),
            compiler_params=pltpu.CompilerParams(use_tc_tiling_on_sc=False),
            scratch_types=[
                pltpu.VMEM((HALF_ROWS * FPAD // IDX_W, IDX_W), jnp.int32),
                pltpu.VMEM((CHUNK * FPAD, EMBED_DIM), jnp.float32),
                pltpu.VMEM((CHUNK * FPAD, EMBED_DIM), jnp.float32),
                pltpu.HBM((CHUNK * FPAD, EMBED_DIM), jnp.float32),
                pltpu.SemaphoreType.DMA,
                pltpu.SemaphoreType.DMA,
                pltpu.SemaphoreType.DMA,
                pltpu.SemaphoreType.DMA,
            ],
        )(x2, table)

    out0 = sc_half(0)
    out1 = sc_half(1)
    ocm = _tc_transpose_out(out0.reshape(BATCH // 2, 8, 128), 0)
    ocm = _tc_transpose_out(out1.reshape(BATCH // 2, 8, 128), 1, prev=ocm)
    return ocm.T
